# baseline (speedup 1.0000x reference)
import functools
import jax
import jax.numpy as jnp
import numpy as np
from jax.experimental import pallas as pl
from jax.experimental.pallas import tpu as pltpu

B, S, D = 1, 2048, 2048
H, NOPE, ROPE = 16, 32, 32
HD = NOPE + ROPE
QL, KVL = 128, 128
NE, NS, NA, IS = 8, 2, 2, 384
EPS = 1e-05

SB = 256
QB = 512


def _rms(x, w):
    ms = jnp.mean(x * x, axis=-1, keepdims=True)
    return w * (x * jax.lax.rsqrt(ms + EPS))


def _rope(x, co, si):
    n = x.shape[0]
    x3 = x.reshape(n, H, ROPE)
    x1 = x3[..., : ROPE // 2]
    x2 = x3[..., ROPE // 2:]
    rot = jnp.concatenate((-x2, x1), axis=-1)
    out = x3 * co[:, None, :] + rot * si[:, None, :]
    return out.reshape(n, H * ROPE)


def _proj_kernel(x_ref, cos_ref, sin_ref, ln1_ref, kvn_ref, qn_ref,
                 wkvd_ref, wuk_ref, wur_ref, wuv_ref, wqd_ref, wuq_ref, wqr_ref,
                 qn_out, qr_out, kn_out, kr_out, v_out):
    xb = x_ref[...]
    xn = _rms(xb, ln1_ref[...])
    ckv = _rms(jnp.dot(xn, wkvd_ref[...], preferred_element_type=jnp.float32),
               kvn_ref[...])
    cq = _rms(jnp.dot(xn, wqd_ref[...], preferred_element_type=jnp.float32),
              qn_ref[...])
    co = cos_ref[...]
    si = sin_ref[...]
    kn_out[...] = jnp.dot(ckv, wuk_ref[...], preferred_element_type=jnp.float32)
    kr = jnp.dot(ckv, wur_ref[...], preferred_element_type=jnp.float32)
    kr_out[...] = _rope(kr, co, si)
    v_out[...] = jnp.dot(ckv, wuv_ref[...], preferred_element_type=jnp.float32)
    qn_out[...] = jnp.dot(cq, wuq_ref[...], preferred_element_type=jnp.float32)
    qr = jnp.dot(cq, wqr_ref[...], preferred_element_type=jnp.float32)
    qr_out[...] = _rope(qr, co, si)


def _attn_kernel(qn_ref, qr_ref, kn_ref, kr_ref, v_ref, ao_ref):
    qn = qn_ref[0]
    qr = qr_ref[0]
    kn = kn_ref[0]
    kr = kr_ref[0]
    dn = (((1,), (1,)), ((), ()))
    s = jax.lax.dot_general(qn, kn, dn, preferred_element_type=jnp.float32)
    s = s + jax.lax.dot_general(qr, kr, dn, preferred_element_type=jnp.float32)
    s = s * (1.0 / np.sqrt(HD))
    m = jnp.max(s, axis=-1, keepdims=True)
    p = jnp.exp(s - m)
    l = jnp.sum(p, axis=-1, keepdims=True)
    ao = jnp.dot(p, v_ref[0], preferred_element_type=jnp.float32)
    ao_ref[0] = ao / l


def _post_kernel(ao_ref, x_ref, wo_ref, ln2_ref, wr_ref, rb_ref,
                 h_out, hn_out, tw_out, ti_out):
    hh = pl.program_id(1)

    @pl.when(hh == 0)
    def _():
        h_out[...] = x_ref[...]

    h_out[...] = h_out[...] + jnp.dot(ao_ref[0], wo_ref[0],
                                      preferred_element_type=jnp.float32)

    @pl.when(hh == H - 1)
    def _():
        h = h_out[...]
        hn = _rms(h, ln2_ref[...])
        hn_out[...] = hn
        logits = jnp.dot(hn, wr_ref[...], preferred_element_type=jnp.float32)
        sig = jax.nn.sigmoid(logits + rb_ref[...])
        w1 = jnp.max(sig, axis=-1, keepdims=True)
        cols = jax.lax.broadcasted_iota(jnp.int32, sig.shape, 1)
        i1 = jnp.min(jnp.where(sig == w1, cols, NE), axis=-1, keepdims=True)
        masked = jnp.where(cols == i1, -jnp.inf, sig)
        w2 = jnp.max(masked, axis=-1, keepdims=True)
        i2 = jnp.min(jnp.where(masked == w2, cols, NE), axis=-1, keepdims=True)
        tot = w1 + w2
        tw_out[...] = jnp.concatenate([w1 / tot, w2 / tot], axis=-1)
        ti_out[...] = jnp.concatenate([i1, i2], axis=-1)


def _shared_kernel(hn_ref, h_ref, g_ref, u_ref, d_ref, base_out):
    hn = hn_ref[...]
    acc = h_ref[...]
    for i in range(NS):
        g = jnp.dot(hn, g_ref[i], preferred_element_type=jnp.float32)
        u = jnp.dot(hn, u_ref[i], preferred_element_type=jnp.float32)
        a = jax.nn.silu(g) * u
        acc = acc + jnp.dot(a, d_ref[i], preferred_element_type=jnp.float32)
    base_out[...] = acc


def _routed_dense_kernel(hn_ref, base_ref, tw_ref, ti_ref, g_ref, u_ref, d_ref,
                         out_ref):
    e = pl.program_id(1)
    hn = hn_ref[...]
    g = jnp.dot(hn, g_ref[0], preferred_element_type=jnp.float32)
    u = jnp.dot(hn, u_ref[0], preferred_element_type=jnp.float32)
    a = jax.nn.silu(g) * u
    eout = jnp.dot(a, d_ref[0], preferred_element_type=jnp.float32)
    sel = (ti_ref[...] == e).astype(jnp.float32)
    scale = jnp.sum(tw_ref[...] * sel, axis=-1, keepdims=True)

    @pl.when(e == 0)
    def _():
        out_ref[...] = base_ref[...] + eout * scale

    @pl.when(e != 0)
    def _():
        out_ref[...] = out_ref[...] + eout * scale


def kernel(x, cos, sin, ln1_w, ln2_w, kv_norm_w, q_norm_w, w_kv_down, w_uk,
           w_ur, w_uv, w_q_down, w_uq, w_qr, w_o, shared_gate, shared_up,
           shared_down, routed_gate, routed_up, routed_down, w_router,
           router_bias):
    xf = x.reshape(S, D)
    ln1 = ln1_w.reshape(1, D)
    ln2 = ln2_w.reshape(1, D)
    kvn = kv_norm_w.reshape(1, KVL)
    qn_w = q_norm_w.reshape(1, QL)
    rb = router_bias.reshape(1, NE)

    nblk = S // SB
    full = lambda shape: pl.BlockSpec(shape, lambda i: (0,) * len(shape))
    rowblk = lambda w: pl.BlockSpec((SB, w), lambda i: (i, 0))

    qn, qr, kn, kr, v = pl.pallas_call(
        _proj_kernel,
        grid=(nblk,),
        in_specs=[
            rowblk(D), rowblk(ROPE), rowblk(ROPE),
            full((1, D)), full((1, KVL)), full((1, QL)),
            full((D, KVL)), full((KVL, H * NOPE)), full((KVL, H * ROPE)),
            full((KVL, H * HD)), full((D, QL)), full((QL, H * NOPE)),
            full((QL, H * ROPE)),
        ],
        out_specs=[rowblk(H * NOPE), rowblk(H * ROPE), rowblk(H * NOPE),
                   rowblk(H * ROPE), rowblk(H * HD)],
        out_shape=[
            jax.ShapeDtypeStruct((S, H * NOPE), jnp.float32),
            jax.ShapeDtypeStruct((S, H * ROPE), jnp.float32),
            jax.ShapeDtypeStruct((S, H * NOPE), jnp.float32),
            jax.ShapeDtypeStruct((S, H * ROPE), jnp.float32),
            jax.ShapeDtypeStruct((S, H * HD), jnp.float32),
        ],
    )(xf, cos, sin, ln1, kvn, qn_w, w_kv_down, w_uk, w_ur, w_uv,
      w_q_down, w_uq, w_qr)

    to3 = lambda a, w: a.reshape(S, H, w).transpose(1, 0, 2)
    qn3, qr3 = to3(qn, NOPE), to3(qr, ROPE)
    kn3, kr3 = to3(kn, NOPE), to3(kr, ROPE)
    v3 = to3(v, HD)

    ao3 = pl.pallas_call(
        _attn_kernel,
        grid=(H, S // QB),
        in_specs=[
            pl.BlockSpec((1, QB, NOPE), lambda h, i: (h, i, 0)),
            pl.BlockSpec((1, QB, ROPE), lambda h, i: (h, i, 0)),
            pl.BlockSpec((1, S, NOPE), lambda h, i: (h, 0, 0)),
            pl.BlockSpec((1, S, ROPE), lambda h, i: (h, 0, 0)),
            pl.BlockSpec((1, S, HD), lambda h, i: (h, 0, 0)),
        ],
        out_specs=pl.BlockSpec((1, QB, HD), lambda h, i: (h, i, 0)),
        out_shape=jax.ShapeDtypeStruct((H, S, HD), jnp.float32),
    )(qn3, qr3, kn3, kr3, v3)

    wo3 = w_o.reshape(H, HD, D)

    h, hn, tw, ti = pl.pallas_call(
        _post_kernel,
        grid=(nblk, H),
        in_specs=[
            pl.BlockSpec((1, SB, HD), lambda i, hh: (hh, i, 0)),
            pl.BlockSpec((SB, D), lambda i, hh: (i, 0)),
            pl.BlockSpec((1, HD, D), lambda i, hh: (hh, 0, 0)),
            pl.BlockSpec((1, D), lambda i, hh: (0, 0)),
            pl.BlockSpec((D, NE), lambda i, hh: (0, 0)),
            pl.BlockSpec((1, NE), lambda i, hh: (0, 0)),
        ],
        out_specs=[
            pl.BlockSpec((SB, D), lambda i, hh: (i, 0)),
            pl.BlockSpec((SB, D), lambda i, hh: (i, 0)),
            pl.BlockSpec((SB, NA), lambda i, hh: (i, 0)),
            pl.BlockSpec((SB, NA), lambda i, hh: (i, 0)),
        ],
        out_shape=[
            jax.ShapeDtypeStruct((S, D), jnp.float32),
            jax.ShapeDtypeStruct((S, D), jnp.float32),
            jax.ShapeDtypeStruct((S, NA), jnp.float32),
            jax.ShapeDtypeStruct((S, NA), jnp.int32),
        ],
    )(ao3, xf, wo3, ln2, w_router, rb)

    base = pl.pallas_call(
        _shared_kernel,
        grid=(nblk,),
        in_specs=[rowblk(D), rowblk(D), full((NS, D, IS)), full((NS, D, IS)),
                  full((NS, IS, D))],
        out_specs=rowblk(D),
        out_shape=jax.ShapeDtypeStruct((S, D), jnp.float32),
    )(hn, h, shared_gate, shared_up, shared_down)

    out = pl.pallas_call(
        _routed_dense_kernel,
        grid=(nblk, NE),
        in_specs=[
            pl.BlockSpec((SB, D), lambda i, e: (i, 0)),
            pl.BlockSpec((SB, D), lambda i, e: (i, 0)),
            pl.BlockSpec((SB, NA), lambda i, e: (i, 0)),
            pl.BlockSpec((SB, NA), lambda i, e: (i, 0)),
            pl.BlockSpec((1, D, IS), lambda i, e: (e, 0, 0)),
            pl.BlockSpec((1, D, IS), lambda i, e: (e, 0, 0)),
            pl.BlockSpec((1, IS, D), lambda i, e: (e, 0, 0)),
        ],
        out_specs=pl.BlockSpec((SB, D), lambda i, e: (i, 0)),
        out_shape=jax.ShapeDtypeStruct((S, D), jnp.float32),
    )(hn, base, tw, ti, routed_gate, routed_up, routed_down)

    onehot = (ti[:, :, None] == jnp.arange(NE)[None, None, :]).astype(jnp.float32)
    expert_usage = jnp.sum(onehot, axis=(0, 1))

    return out.reshape(B, S, D), expert_usage
